# baseline (speedup 1.0000x reference)
import functools

import jax
import jax.numpy as jnp
from jax import lax
from jax.experimental import pallas as pl
from jax.experimental.pallas import tpu as pltpu
from jax.experimental.pallas import tpu_sc as plsc

N_ATOMS = 10000
N_EDGES = 160000
HIDDEN = 16
OUT = 16
NUM_IRREPS = 16

NC = 2
NS = 16
L = 16
NW = NC * NS

E_PAD = 163840
C = 128
NCH0 = 54
NCH1 = (E_PAD // (NS * C)) - NCH0
EPW0 = NCH0 * C
EPW1 = NCH1 * C

HALF = 5056
NROWS = 2 * HALF
EPS = E_PAD // NS
D = 512
NDCH = EPS // D

_SQ3 = 1.7320508075688772
_C4 = 3.872983346207417
_C6 = 1.118033988749895
_C8 = 1.9364916731037085
_C9 = 2.091650066335189
_C10 = 10.246950765959598
_C11 = 1.620185174601965
_C12 = 1.3228756555322954
_C14 = 5.123475382979799


def _mm_body(nf_ref, b_ref, o_ref):
    o_ref[:, :] = jnp.dot(nf_ref[:, :], b_ref[:, :],
                          preferred_element_type=jnp.float32)


def _reduce_body(parts_ref, b_ref, o_ref):
    o_ref[:] = jnp.sum(parts_ref[:, :], axis=0) + b_ref[:]


def _sh_scalars(x, y, z):
    xx = x * x
    yy = y * y
    zz = z * z
    r2 = xx + yy + zz
    xy = x * y
    yz = y * z
    xz = x * z
    return (
        None,
        _SQ3 * x,
        _SQ3 * y,
        _SQ3 * z,
        _C4 * xy,
        _C4 * yz,
        _C6 * (3.0 * zz - r2),
        _C4 * xz,
        _C8 * (xx - yy),
        _C9 * y * (3.0 * xx - yy),
        _C10 * xy * z,
        _C11 * y * (5.0 * zz - r2),
        _C12 * z * (5.0 * zz - 3.0 * r2),
        _C11 * x * (5.0 * zz - r2),
        _C14 * z * (xx - yy),
        _C9 * x * (xx - 3.0 * yy),
    )


@functools.partial(
    pl.kernel,
    out_type=jax.ShapeDtypeStruct((E_PAD * OUT,), jnp.float32),
    mesh=plsc.VectorSubcoreMesh(core_axis_name="c", subcore_axis_name="s"),
    scratch_types=[
        pltpu.VMEM((C,), jnp.int32),
        pltpu.VMEM((C,), jnp.int32),
        pltpu.VMEM((3 * C,), jnp.float32),
        pltpu.VMEM((3 * C,), jnp.float32),
        pltpu.VMEM((C, NUM_IRREPS * OUT), jnp.float32),
        pltpu.VMEM((C, NUM_IRREPS * OUT), jnp.float32),
        pltpu.VMEM((C * OUT,), jnp.float32),
        pltpu.VMEM((C * OUT,), jnp.float32),
        pltpu.SemaphoreType.DMA,
        pltpu.SemaphoreType.DMA,
        pltpu.SemaphoreType.DMA,
        pltpu.SemaphoreType.DMA,
        pltpu.SemaphoreType.DMA,
        pltpu.SemaphoreType.DMA,
    ],
)
def _edge_kernel(p_hbm, src_hbm, xyz_hbm, z_out,
                 idx_a, idx_b, xyz_a, xyz_b, rows_a, rows_b,
                 zbuf_a, zbuf_b, si_a, si_b, sg_a, sg_b, sw_a, sw_b):
    cid = lax.axis_index("c")
    sid = lax.axis_index("s")
    base = jnp.where(cid == 0, sid * EPW0, NS * EPW0 + sid * EPW1)
    nchunk = jnp.where(cid == 0, NCH0, NCH1)

    bufs = (
        (idx_a, xyz_a, rows_a, zbuf_a, si_a, sg_a, sw_a),
        (idx_b, xyz_b, rows_b, zbuf_b, si_b, sg_b, sw_b),
    )

    def start_in(ci, b):
        off = base + ci * C
        pltpu.async_copy(src_hbm.at[pl.ds(off, C)], b[0], b[4])
        pltpu.async_copy(xyz_hbm.at[pl.ds(off * 3, 3 * C)], b[1], b[4])

    def wait_in(b):
        pltpu.make_async_copy(src_hbm.at[pl.ds(0, C)], b[0], b[4]).wait()
        pltpu.make_async_copy(xyz_hbm.at[pl.ds(0, 3 * C)], b[1], b[4]).wait()

    def start_gather(b):
        pltpu.async_copy(p_hbm.at[b[0]], b[2], b[5])

    def wait_gather(b):
        pltpu.make_async_copy(p_hbm.at[b[0]], b[2], b[5]).wait()

    def start_wb(ci, b):
        off = base + ci * C
        pltpu.async_copy(b[3], z_out.at[pl.ds(off * OUT, C * OUT)], b[6])

    def wait_wb(b):
        pltpu.make_async_copy(b[3], z_out.at[pl.ds(0, C * OUT)], b[6]).wait()

    def compute(b):
        xyz, rows_v, zbuf = b[1], b[2], b[3]

        def _group(g, carry2):
            xs = xyz[pl.ds(g * L, L)]
            ys = xyz[pl.ds(C + g * L, L)]
            zs = xyz[pl.ds(2 * C + g * L, L)]
            for el in range(L):
                e = g * L + el
                sh = _sh_scalars(xs[el], ys[el], zs[el])
                terms = [rows_v[e, pl.ds(0, L)]]
                for i in range(1, NUM_IRREPS):
                    terms.append(sh[i] * rows_v[e, pl.ds(i * L, L)])
                while len(terms) > 1:
                    terms = [terms[j] + terms[j + 1]
                             for j in range(0, len(terms) - 1, 2)] + (
                                 [terms[-1]] if len(terms) % 2 else [])
                zbuf[pl.ds(e * OUT, OUT)] = terms[0]
            return carry2
        lax.fori_loop(0, C // L, _group, 0)

    start_in(0, bufs[0])
    wait_in(bufs[0])
    start_gather(bufs[0])
    start_in(1, bufs[1])

    def _pair(k, carry):
        for p in range(2):
            ci = k * 2 + p
            this = bufs[p]
            other = bufs[1 - p]
            @pl.when(ci + 1 < nchunk)
            def _():
                wait_in(other)
                start_gather(other)
            wait_gather(this)

            @pl.when(ci >= 2)
            def _():
                wait_wb(this)
            compute(this)
            start_wb(ci, this)

            @pl.when(ci + 2 < nchunk)
            def _():
                start_in(ci + 2, this)
        return carry

    lax.fori_loop(0, nchunk // 2, _pair, 0)
    wait_wb(bufs[0])
    wait_wb(bufs[1])


@functools.partial(
    pl.kernel,
    out_type=jax.ShapeDtypeStruct((NS, NROWS * OUT), jnp.float32),
    mesh=plsc.VectorSubcoreMesh(core_axis_name="c", subcore_axis_name="s"),
    scratch_types=[
        pltpu.VMEM((D,), jnp.int32),
        pltpu.VMEM((D,), jnp.int32),
        pltpu.VMEM((D * OUT,), jnp.float32),
        pltpu.VMEM((D * OUT,), jnp.float32),
        pltpu.VMEM(((HALF + L) * OUT,), jnp.float32),
        pltpu.SemaphoreType.DMA,
        pltpu.SemaphoreType.DMA,
    ],
)
def _scatter_kernel(tgt_hbm, z_hbm, out_hbm, tgt_a, tgt_b, zc_a, zc_b,
                    table, sm_a, sm_b):
    cid = lax.axis_index("c")
    sid = lax.axis_index("s")
    nbase = cid * HALF
    ebase = sid * EPS

    bufs = ((tgt_a, zc_a, sm_a), (tgt_b, zc_b, sm_b))

    def start_in(di, b):
        off = ebase + di * D
        pltpu.async_copy(tgt_hbm.at[pl.ds(off, D)], b[0], b[2])
        pltpu.async_copy(z_hbm.at[pl.ds(off * OUT, D * OUT)], b[1], b[2])

    def wait_in(b):
        pltpu.make_async_copy(tgt_hbm.at[pl.ds(0, D)], b[0], b[2]).wait()
        pltpu.make_async_copy(z_hbm.at[pl.ds(0, D * OUT)], b[1], b[2]).wait()

    def accumulate(b):
        tgt_v, zc = b[0], b[1]

        def _group(g, carry2):
            tg = tgt_v[pl.ds(g * L, L)]
            local = tg - nbase
            in_half = (local >= 0) & (local < HALF)
            offs = jnp.where(in_half, local, HALF) * OUT
            for el in range(L):
                e = g * L + el
                plsc.addupdate(table.at[pl.ds(offs[el], OUT)],
                               zc[pl.ds(e * OUT, OUT)])
            return carry2
        lax.fori_loop(0, D // L, _group, 0)

    def _zero(i, carry):
        table[pl.ds(i * L, L)] = jnp.zeros((L,), jnp.float32)
        return carry
    start_in(0, bufs[0])
    start_in(1, bufs[1])
    lax.fori_loop(0, (HALF + L) * OUT // L, _zero, 0)

    def _pair(k, carry):
        for p in range(2):
            di = k * 2 + p
            this = bufs[p]
            wait_in(this)
            accumulate(this)

            @pl.when(di + 2 < NDCH)
            def _():
                start_in(di + 2, this)
        return carry

    lax.fori_loop(0, NDCH // 2, _pair, 0)

    pltpu.sync_copy(table.at[pl.ds(0, HALF * OUT)],
                    out_hbm.at[sid, pl.ds(nbase * OUT, HALF * OUT)])


def kernel(node_features, edge_vectors, edge_index, tp_weights, W_out, b_out):
    W3 = W_out.reshape(OUT, OUT, NUM_IRREPS)
    A = jnp.einsum('iho,poi->ihp', tp_weights, W3)
    Bmat = jnp.einsum('ihp,ij->hijp', A,
                      jnp.eye(NUM_IRREPS, dtype=jnp.float32))
    Bmat = Bmat.reshape(HIDDEN * NUM_IRREPS, NUM_IRREPS * OUT)

    nf_flat = node_features.reshape(N_ATOMS, HIDDEN * NUM_IRREPS)
    p_flat = pl.pallas_call(
        _mm_body,
        out_shape=jax.ShapeDtypeStruct((N_ATOMS, NUM_IRREPS * OUT),
                                       jnp.float32),
    )(nf_flat, Bmat)

    pad = E_PAD - N_EDGES
    srcs = jnp.concatenate(
        [edge_index[0].astype(jnp.int32), jnp.zeros((pad,), jnp.int32)])
    tgts = jnp.concatenate(
        [edge_index[1].astype(jnp.int32),
         jnp.full((pad,), N_ATOMS, jnp.int32)])
    ev_pad = jnp.concatenate(
        [edge_vectors, jnp.zeros((pad, 3), jnp.float32)])
    xyz_pk = ev_pad.T.reshape(3, E_PAD // C, C).transpose(1, 0, 2).reshape(-1)

    z_rows = _edge_kernel(p_flat, srcs, xyz_pk)
    parts = _scatter_kernel(tgts, z_rows)

    out = pl.pallas_call(
        _reduce_body,
        out_shape=jax.ShapeDtypeStruct((NROWS * OUT,), jnp.float32),
    )(parts, jnp.tile(b_out, NROWS))

    return out.reshape(NROWS, OUT)[:N_ATOMS]


# speedup vs baseline: 1.0042x; 1.0042x over previous
import functools

import jax
import jax.numpy as jnp
from jax import lax
from jax.experimental import pallas as pl
from jax.experimental.pallas import tpu as pltpu
from jax.experimental.pallas import tpu_sc as plsc

N_ATOMS = 10000
N_EDGES = 160000
HIDDEN = 16
OUT = 16
NUM_IRREPS = 16

NC = 2
NS = 16
L = 16
NW = NC * NS

E_PAD = 163840
C = 64
NCH0 = 110
NCH1 = (E_PAD // (NS * C)) - NCH0
EPW0 = NCH0 * C
EPW1 = NCH1 * C

HALF = 5056
NROWS = 2 * HALF
EPS = E_PAD // NS
D = 512
NDCH = EPS // D

_SQ3 = 1.7320508075688772
_C4 = 3.872983346207417
_C6 = 1.118033988749895
_C8 = 1.9364916731037085
_C9 = 2.091650066335189
_C10 = 10.246950765959598
_C11 = 1.620185174601965
_C12 = 1.3228756555322954
_C14 = 5.123475382979799


def _mm_body(nf_ref, b_ref, o_ref):
    o_ref[:, :] = jnp.dot(nf_ref[:, :], b_ref[:, :],
                          preferred_element_type=jnp.float32)


def _reduce_body(parts_ref, b_ref, o_ref):
    o_ref[:] = jnp.sum(parts_ref[:, :], axis=0) + b_ref[:]


def _sh_scalars(x, y, z):
    xx = x * x
    yy = y * y
    zz = z * z
    r2 = xx + yy + zz
    xy = x * y
    yz = y * z
    xz = x * z
    return (
        None,
        _SQ3 * x,
        _SQ3 * y,
        _SQ3 * z,
        _C4 * xy,
        _C4 * yz,
        _C6 * (3.0 * zz - r2),
        _C4 * xz,
        _C8 * (xx - yy),
        _C9 * y * (3.0 * xx - yy),
        _C10 * xy * z,
        _C11 * y * (5.0 * zz - r2),
        _C12 * z * (5.0 * zz - 3.0 * r2),
        _C11 * x * (5.0 * zz - r2),
        _C14 * z * (xx - yy),
        _C9 * x * (xx - 3.0 * yy),
    )


@functools.partial(
    pl.kernel,
    out_type=jax.ShapeDtypeStruct((E_PAD * OUT,), jnp.float32),
    mesh=plsc.VectorSubcoreMesh(core_axis_name="c", subcore_axis_name="s"),
    scratch_types=[
        pltpu.VMEM((C,), jnp.int32),
        pltpu.VMEM((C,), jnp.int32),
        pltpu.VMEM((3 * C,), jnp.float32),
        pltpu.VMEM((3 * C,), jnp.float32),
        pltpu.VMEM((C, NUM_IRREPS * OUT), jnp.float32),
        pltpu.VMEM((C, NUM_IRREPS * OUT), jnp.float32),
        pltpu.VMEM((C * OUT,), jnp.float32),
        pltpu.VMEM((C * OUT,), jnp.float32),
        pltpu.SemaphoreType.DMA,
        pltpu.SemaphoreType.DMA,
        pltpu.SemaphoreType.DMA,
        pltpu.SemaphoreType.DMA,
        pltpu.SemaphoreType.DMA,
        pltpu.SemaphoreType.DMA,
    ],
)
def _edge_kernel(p_hbm, src_hbm, xyz_hbm, z_out,
                 idx_a, idx_b, xyz_a, xyz_b, rows_a, rows_b,
                 zbuf_a, zbuf_b, si_a, si_b, sg_a, sg_b, sw_a, sw_b):
    cid = lax.axis_index("c")
    sid = lax.axis_index("s")
    base = jnp.where(cid == 0, sid * EPW0, NS * EPW0 + sid * EPW1)
    nchunk = jnp.where(cid == 0, NCH0, NCH1)

    bufs = (
        (idx_a, xyz_a, rows_a, zbuf_a, si_a, sg_a, sw_a),
        (idx_b, xyz_b, rows_b, zbuf_b, si_b, sg_b, sw_b),
    )

    def start_in(ci, b):
        off = base + ci * C
        pltpu.async_copy(src_hbm.at[pl.ds(off, C)], b[0], b[4])
        pltpu.async_copy(xyz_hbm.at[pl.ds(off * 3, 3 * C)], b[1], b[4])

    def wait_in(b):
        pltpu.make_async_copy(src_hbm.at[pl.ds(0, C)], b[0], b[4]).wait()
        pltpu.make_async_copy(xyz_hbm.at[pl.ds(0, 3 * C)], b[1], b[4]).wait()

    def start_gather(b):
        pltpu.async_copy(p_hbm.at[b[0]], b[2], b[5])

    def wait_gather(b):
        pltpu.make_async_copy(p_hbm.at[b[0]], b[2], b[5]).wait()

    def start_wb(ci, b):
        off = base + ci * C
        pltpu.async_copy(b[3], z_out.at[pl.ds(off * OUT, C * OUT)], b[6])

    def wait_wb(b):
        pltpu.make_async_copy(b[3], z_out.at[pl.ds(0, C * OUT)], b[6]).wait()

    def compute(b):
        xyz, rows_v, zbuf = b[1], b[2], b[3]

        def _group(g, carry2):
            xs = xyz[pl.ds(g * L, L)]
            ys = xyz[pl.ds(C + g * L, L)]
            zs = xyz[pl.ds(2 * C + g * L, L)]
            for el in range(L):
                e = g * L + el
                sh = _sh_scalars(xs[el], ys[el], zs[el])
                terms = [rows_v[e, pl.ds(0, L)]]
                for i in range(1, NUM_IRREPS):
                    terms.append(sh[i] * rows_v[e, pl.ds(i * L, L)])
                while len(terms) > 1:
                    terms = [terms[j] + terms[j + 1]
                             for j in range(0, len(terms) - 1, 2)] + (
                                 [terms[-1]] if len(terms) % 2 else [])
                zbuf[pl.ds(e * OUT, OUT)] = terms[0]
            return carry2
        lax.fori_loop(0, C // L, _group, 0)

    start_in(0, bufs[0])
    wait_in(bufs[0])
    start_gather(bufs[0])
    start_in(1, bufs[1])

    def _pair(k, carry):
        for p in range(2):
            ci = k * 2 + p
            this = bufs[p]
            other = bufs[1 - p]
            @pl.when(ci + 1 < nchunk)
            def _():
                wait_in(other)
                start_gather(other)
            wait_gather(this)

            @pl.when(ci >= 2)
            def _():
                wait_wb(this)
            compute(this)
            start_wb(ci, this)

            @pl.when(ci + 2 < nchunk)
            def _():
                start_in(ci + 2, this)
        return carry

    lax.fori_loop(0, nchunk // 2, _pair, 0)
    wait_wb(bufs[0])
    wait_wb(bufs[1])


@functools.partial(
    pl.kernel,
    out_type=jax.ShapeDtypeStruct((NS, NROWS * OUT), jnp.float32),
    mesh=plsc.VectorSubcoreMesh(core_axis_name="c", subcore_axis_name="s"),
    scratch_types=[
        pltpu.VMEM((D,), jnp.int32),
        pltpu.VMEM((D,), jnp.int32),
        pltpu.VMEM((D * OUT,), jnp.float32),
        pltpu.VMEM((D * OUT,), jnp.float32),
        pltpu.VMEM(((HALF + L) * OUT,), jnp.float32),
        pltpu.SemaphoreType.DMA,
        pltpu.SemaphoreType.DMA,
    ],
)
def _scatter_kernel(tgt_hbm, z_hbm, out_hbm, tgt_a, tgt_b, zc_a, zc_b,
                    table, sm_a, sm_b):
    cid = lax.axis_index("c")
    sid = lax.axis_index("s")
    nbase = cid * HALF
    ebase = sid * EPS

    bufs = ((tgt_a, zc_a, sm_a), (tgt_b, zc_b, sm_b))

    def start_in(di, b):
        off = ebase + di * D
        pltpu.async_copy(tgt_hbm.at[pl.ds(off, D)], b[0], b[2])
        pltpu.async_copy(z_hbm.at[pl.ds(off * OUT, D * OUT)], b[1], b[2])

    def wait_in(b):
        pltpu.make_async_copy(tgt_hbm.at[pl.ds(0, D)], b[0], b[2]).wait()
        pltpu.make_async_copy(z_hbm.at[pl.ds(0, D * OUT)], b[1], b[2]).wait()

    def accumulate(b):
        tgt_v, zc = b[0], b[1]

        def _group(g, carry2):
            tg = tgt_v[pl.ds(g * L, L)]
            local = tg - nbase
            in_half = (local >= 0) & (local < HALF)
            offs = jnp.where(in_half, local, HALF) * OUT
            for el in range(L):
                e = g * L + el
                plsc.addupdate(table.at[pl.ds(offs[el], OUT)],
                               zc[pl.ds(e * OUT, OUT)])
            return carry2
        lax.fori_loop(0, D // L, _group, 0)

    def _zero(i, carry):
        table[pl.ds(i * L, L)] = jnp.zeros((L,), jnp.float32)
        return carry
    start_in(0, bufs[0])
    start_in(1, bufs[1])
    lax.fori_loop(0, (HALF + L) * OUT // L, _zero, 0)

    def _pair(k, carry):
        for p in range(2):
            di = k * 2 + p
            this = bufs[p]
            wait_in(this)
            accumulate(this)

            @pl.when(di + 2 < NDCH)
            def _():
                start_in(di + 2, this)
        return carry

    lax.fori_loop(0, NDCH // 2, _pair, 0)

    pltpu.sync_copy(table.at[pl.ds(0, HALF * OUT)],
                    out_hbm.at[sid, pl.ds(nbase * OUT, HALF * OUT)])


def kernel(node_features, edge_vectors, edge_index, tp_weights, W_out, b_out):
    W3 = W_out.reshape(OUT, OUT, NUM_IRREPS)
    A = jnp.einsum('iho,poi->ihp', tp_weights, W3)
    Bmat = jnp.einsum('ihp,ij->hijp', A,
                      jnp.eye(NUM_IRREPS, dtype=jnp.float32))
    Bmat = Bmat.reshape(HIDDEN * NUM_IRREPS, NUM_IRREPS * OUT)

    nf_flat = node_features.reshape(N_ATOMS, HIDDEN * NUM_IRREPS)
    p_flat = pl.pallas_call(
        _mm_body,
        out_shape=jax.ShapeDtypeStruct((N_ATOMS, NUM_IRREPS * OUT),
                                       jnp.float32),
    )(nf_flat, Bmat)

    pad = E_PAD - N_EDGES
    srcs = jnp.concatenate(
        [edge_index[0].astype(jnp.int32), jnp.zeros((pad,), jnp.int32)])
    tgts = jnp.concatenate(
        [edge_index[1].astype(jnp.int32),
         jnp.full((pad,), N_ATOMS, jnp.int32)])
    ev_pad = jnp.concatenate(
        [edge_vectors, jnp.zeros((pad, 3), jnp.float32)])
    xyz_pk = ev_pad.T.reshape(3, E_PAD // C, C).transpose(1, 0, 2).reshape(-1)

    z_rows = _edge_kernel(p_flat, srcs, xyz_pk)
    parts = _scatter_kernel(tgts, z_rows)

    out = pl.pallas_call(
        _reduce_body,
        out_shape=jax.ShapeDtypeStruct((NROWS * OUT,), jnp.float32),
    )(parts, jnp.tile(b_out, NROWS))

    return out.reshape(NROWS, OUT)[:N_ATOMS]


# speedup vs baseline: 1.0407x; 1.0363x over previous
import functools

import jax
import jax.numpy as jnp
from jax import lax
from jax.experimental import pallas as pl
from jax.experimental.pallas import tpu as pltpu
from jax.experimental.pallas import tpu_sc as plsc

N_ATOMS = 10000
N_EDGES = 160000
HIDDEN = 16
OUT = 16
NUM_IRREPS = 16

NC = 2
NS = 16
L = 16
NW = NC * NS

E_PAD = 163840
C = 80
NCH0 = 88
NCH1 = (E_PAD // (NS * C)) - NCH0
EPW0 = NCH0 * C
EPW1 = NCH1 * C

HALF = 5056
NROWS = 2 * HALF
EPS = E_PAD // NS
D = 512
NDCH = EPS // D

_SQ3 = 1.7320508075688772
_C4 = 3.872983346207417
_C6 = 1.118033988749895
_C8 = 1.9364916731037085
_C9 = 2.091650066335189
_C10 = 10.246950765959598
_C11 = 1.620185174601965
_C12 = 1.3228756555322954
_C14 = 5.123475382979799


def _mm_body(nf_ref, b_ref, o_ref):
    o_ref[:, :] = jnp.dot(nf_ref[:, :], b_ref[:, :],
                          preferred_element_type=jnp.float32)


def _reduce_body(parts_ref, b_ref, o_ref):
    o_ref[:] = jnp.sum(parts_ref[:, :], axis=0) + b_ref[:]


def _sh_scalars(x, y, z):
    xx = x * x
    yy = y * y
    zz = z * z
    r2 = xx + yy + zz
    xy = x * y
    yz = y * z
    xz = x * z
    return (
        None,
        _SQ3 * x,
        _SQ3 * y,
        _SQ3 * z,
        _C4 * xy,
        _C4 * yz,
        _C6 * (3.0 * zz - r2),
        _C4 * xz,
        _C8 * (xx - yy),
        _C9 * y * (3.0 * xx - yy),
        _C10 * xy * z,
        _C11 * y * (5.0 * zz - r2),
        _C12 * z * (5.0 * zz - 3.0 * r2),
        _C11 * x * (5.0 * zz - r2),
        _C14 * z * (xx - yy),
        _C9 * x * (xx - 3.0 * yy),
    )


@functools.partial(
    pl.kernel,
    out_type=jax.ShapeDtypeStruct((E_PAD * OUT,), jnp.float32),
    mesh=plsc.VectorSubcoreMesh(core_axis_name="c", subcore_axis_name="s"),
    scratch_types=[
        pltpu.VMEM((C,), jnp.int32),
        pltpu.VMEM((C,), jnp.int32),
        pltpu.VMEM((3 * C,), jnp.float32),
        pltpu.VMEM((3 * C,), jnp.float32),
        pltpu.VMEM((C, NUM_IRREPS * OUT), jnp.float32),
        pltpu.VMEM((C, NUM_IRREPS * OUT), jnp.float32),
        pltpu.VMEM((C * OUT,), jnp.float32),
        pltpu.VMEM((C * OUT,), jnp.float32),
        pltpu.SemaphoreType.DMA,
        pltpu.SemaphoreType.DMA,
        pltpu.SemaphoreType.DMA,
        pltpu.SemaphoreType.DMA,
        pltpu.SemaphoreType.DMA,
        pltpu.SemaphoreType.DMA,
    ],
)
def _edge_kernel(p_hbm, src_hbm, xyz_hbm, z_out,
                 idx_a, idx_b, xyz_a, xyz_b, rows_a, rows_b,
                 zbuf_a, zbuf_b, si_a, si_b, sg_a, sg_b, sw_a, sw_b):
    cid = lax.axis_index("c")
    sid = lax.axis_index("s")
    base = jnp.where(cid == 0, sid * EPW0, NS * EPW0 + sid * EPW1)
    nchunk = jnp.where(cid == 0, NCH0, NCH1)

    bufs = (
        (idx_a, xyz_a, rows_a, zbuf_a, si_a, sg_a, sw_a),
        (idx_b, xyz_b, rows_b, zbuf_b, si_b, sg_b, sw_b),
    )

    def start_in(ci, b):
        off = base + ci * C
        pltpu.async_copy(src_hbm.at[pl.ds(off, C)], b[0], b[4])
        pltpu.async_copy(xyz_hbm.at[pl.ds(off * 3, 3 * C)], b[1], b[4])

    def wait_in(b):
        pltpu.make_async_copy(src_hbm.at[pl.ds(0, C)], b[0], b[4]).wait()
        pltpu.make_async_copy(xyz_hbm.at[pl.ds(0, 3 * C)], b[1], b[4]).wait()

    def start_gather(b):
        pltpu.async_copy(p_hbm.at[b[0]], b[2], b[5])

    def wait_gather(b):
        pltpu.make_async_copy(p_hbm.at[b[0]], b[2], b[5]).wait()

    def start_wb(ci, b):
        off = base + ci * C
        pltpu.async_copy(b[3], z_out.at[pl.ds(off * OUT, C * OUT)], b[6])

    def wait_wb(b):
        pltpu.make_async_copy(b[3], z_out.at[pl.ds(0, C * OUT)], b[6]).wait()

    def compute(b):
        xyz, rows_v, zbuf = b[1], b[2], b[3]

        def _group(g, carry2):
            xs = xyz[pl.ds(g * L, L)]
            ys = xyz[pl.ds(C + g * L, L)]
            zs = xyz[pl.ds(2 * C + g * L, L)]
            for el in range(L):
                e = g * L + el
                sh = _sh_scalars(xs[el], ys[el], zs[el])
                terms = [rows_v[e, pl.ds(0, L)]]
                for i in range(1, NUM_IRREPS):
                    terms.append(sh[i] * rows_v[e, pl.ds(i * L, L)])
                while len(terms) > 1:
                    terms = [terms[j] + terms[j + 1]
                             for j in range(0, len(terms) - 1, 2)] + (
                                 [terms[-1]] if len(terms) % 2 else [])
                zbuf[pl.ds(e * OUT, OUT)] = terms[0]
            return carry2
        lax.fori_loop(0, C // L, _group, 0)

    start_in(0, bufs[0])
    wait_in(bufs[0])
    start_gather(bufs[0])
    start_in(1, bufs[1])

    def _pair(k, carry):
        for p in range(2):
            ci = k * 2 + p
            this = bufs[p]
            other = bufs[1 - p]
            @pl.when(ci + 1 < nchunk)
            def _():
                wait_in(other)
                start_gather(other)
            wait_gather(this)

            @pl.when(ci >= 2)
            def _():
                wait_wb(this)
            compute(this)
            start_wb(ci, this)

            @pl.when(ci + 2 < nchunk)
            def _():
                start_in(ci + 2, this)
        return carry

    lax.fori_loop(0, nchunk // 2, _pair, 0)
    wait_wb(bufs[0])
    wait_wb(bufs[1])


@functools.partial(
    pl.kernel,
    out_type=jax.ShapeDtypeStruct((NS, NROWS * OUT), jnp.float32),
    mesh=plsc.VectorSubcoreMesh(core_axis_name="c", subcore_axis_name="s"),
    scratch_types=[
        pltpu.VMEM((D,), jnp.int32),
        pltpu.VMEM((D,), jnp.int32),
        pltpu.VMEM((D * OUT,), jnp.float32),
        pltpu.VMEM((D * OUT,), jnp.float32),
        pltpu.VMEM(((HALF + L) * OUT,), jnp.float32),
        pltpu.SemaphoreType.DMA,
        pltpu.SemaphoreType.DMA,
    ],
)
def _scatter_kernel(tgt_hbm, z_hbm, out_hbm, tgt_a, tgt_b, zc_a, zc_b,
                    table, sm_a, sm_b):
    cid = lax.axis_index("c")
    sid = lax.axis_index("s")
    nbase = cid * HALF
    ebase = sid * EPS

    bufs = ((tgt_a, zc_a, sm_a), (tgt_b, zc_b, sm_b))

    def start_in(di, b):
        off = ebase + di * D
        pltpu.async_copy(tgt_hbm.at[pl.ds(off, D)], b[0], b[2])
        pltpu.async_copy(z_hbm.at[pl.ds(off * OUT, D * OUT)], b[1], b[2])

    def wait_in(b):
        pltpu.make_async_copy(tgt_hbm.at[pl.ds(0, D)], b[0], b[2]).wait()
        pltpu.make_async_copy(z_hbm.at[pl.ds(0, D * OUT)], b[1], b[2]).wait()

    def accumulate(b):
        tgt_v, zc = b[0], b[1]

        def _group(g, carry2):
            tg = tgt_v[pl.ds(g * L, L)]
            local = tg - nbase
            in_half = (local >= 0) & (local < HALF)
            offs = jnp.where(in_half, local, HALF) * OUT
            for el in range(L):
                e = g * L + el
                plsc.addupdate(table.at[pl.ds(offs[el], OUT)],
                               zc[pl.ds(e * OUT, OUT)])
            return carry2
        lax.fori_loop(0, D // L, _group, 0)

    def _zero(i, carry):
        table[pl.ds(i * L, L)] = jnp.zeros((L,), jnp.float32)
        return carry
    start_in(0, bufs[0])
    start_in(1, bufs[1])
    lax.fori_loop(0, (HALF + L) * OUT // L, _zero, 0)

    def _pair(k, carry):
        for p in range(2):
            di = k * 2 + p
            this = bufs[p]
            wait_in(this)
            accumulate(this)

            @pl.when(di + 2 < NDCH)
            def _():
                start_in(di + 2, this)
        return carry

    lax.fori_loop(0, NDCH // 2, _pair, 0)

    pltpu.sync_copy(table.at[pl.ds(0, HALF * OUT)],
                    out_hbm.at[sid, pl.ds(nbase * OUT, HALF * OUT)])


def kernel(node_features, edge_vectors, edge_index, tp_weights, W_out, b_out):
    W3 = W_out.reshape(OUT, OUT, NUM_IRREPS)
    A = jnp.einsum('iho,poi->ihp', tp_weights, W3)
    Bmat = jnp.einsum('ihp,ij->hijp', A,
                      jnp.eye(NUM_IRREPS, dtype=jnp.float32))
    Bmat = Bmat.reshape(HIDDEN * NUM_IRREPS, NUM_IRREPS * OUT)

    nf_flat = node_features.reshape(N_ATOMS, HIDDEN * NUM_IRREPS)
    p_flat = pl.pallas_call(
        _mm_body,
        out_shape=jax.ShapeDtypeStruct((N_ATOMS, NUM_IRREPS * OUT),
                                       jnp.float32),
    )(nf_flat, Bmat)

    pad = E_PAD - N_EDGES
    srcs = jnp.concatenate(
        [edge_index[0].astype(jnp.int32), jnp.zeros((pad,), jnp.int32)])
    tgts = jnp.concatenate(
        [edge_index[1].astype(jnp.int32),
         jnp.full((pad,), N_ATOMS, jnp.int32)])
    ev_pad = jnp.concatenate(
        [edge_vectors, jnp.zeros((pad, 3), jnp.float32)])
    xyz_pk = ev_pad.T.reshape(3, E_PAD // C, C).transpose(1, 0, 2).reshape(-1)

    z_rows = _edge_kernel(p_flat, srcs, xyz_pk)
    parts = _scatter_kernel(tgts, z_rows)

    out = pl.pallas_call(
        _reduce_body,
        out_shape=jax.ShapeDtypeStruct((NROWS * OUT,), jnp.float32),
    )(parts, jnp.tile(b_out, NROWS))

    return out.reshape(NROWS, OUT)[:N_ATOMS]
